# baseline (speedup 1.0000x reference)
import functools

import jax
import jax.numpy as jnp
from jax import lax
from jax.experimental import pallas as pl
from jax.experimental.pallas import tpu as pltpu
from jax.experimental.pallas import tpu_sc as plsc

N = 10000
E = 320000
D = 128

NC = 2
NS = 16
NW = NC * NS
K = 128
CB = 8
SB = 10
EP = NW * SB * CB * K
NP = 10240
RPT = NP // NS


def _sc_segment_sum(x, ei, zrows):
    mesh = plsc.VectorSubcoreMesh(core_axis_name="c", subcore_axis_name="s")

    @functools.partial(
        pl.kernel,
        out_type=jax.ShapeDtypeStruct((NC, NP, D), jnp.float32),
        mesh=mesh,
        scratch_types=[
            pltpu.VMEM((CB, K), jnp.int32),
            pltpu.VMEM((CB, K), jnp.int32),
            pltpu.VMEM((2, K, D), jnp.float32),
            pltpu.VMEM_SHARED((NP, D), jnp.float32),
            pltpu.SemaphoreType.DMA,
            pltpu.SemaphoreType.DMA,
        ],
    )
    def agg_kernel(x_hbm, ei_hbm, z_hbm, out_hbm, srcv, dstv, buf, agg_sh,
                   gsem, ssem):
        cid = lax.axis_index("c")
        sid = lax.axis_index("s")
        wid = cid * NS + sid
        base = sid * RPT
        pltpu.sync_copy(z_hbm, agg_sh.at[pl.ds(base, RPT)])
        plsc.subcore_barrier()

        def superblock(s, carry):
            pltpu.sync_copy(ei_hbm.at[0, wid, s], srcv)
            pltpu.sync_copy(ei_hbm.at[1, wid, s], dstv)
            pltpu.async_copy(x_hbm.at[srcv.at[0]], buf.at[0], gsem)

            def step(j, c):
                slot = lax.rem(j, 2)
                nslot = lax.rem(j + 1, 2)
                pltpu.make_async_copy(x_hbm.at[srcv.at[j]], buf.at[slot],
                                      gsem).wait()
                jn = lax.min(j + 1, CB - 1)
                pltpu.async_copy(x_hbm.at[srcv.at[jn]], buf.at[nslot], gsem)
                pltpu.sync_copy(buf.at[slot], agg_sh.at[dstv.at[j]],
                                add=True)
                return c

            lax.fori_loop(0, CB, step, 0)
            pltpu.make_async_copy(x_hbm.at[srcv.at[CB - 1]],
                                  buf.at[CB % 2], gsem).wait()
            return carry

        lax.fori_loop(0, SB, superblock, 0)
        plsc.subcore_barrier()
        pltpu.sync_copy(agg_sh.at[pl.ds(base, RPT)],
                        out_hbm.at[cid, pl.ds(base, RPT)])

    return agg_kernel(x, ei, zrows)


MB = 512


def _mlp_body(x_ref, p_ref, w1_ref, b1_ref, w2_ref, b2_ref, o_ref):
    h = x_ref[...] + p_ref[0] + p_ref[1]
    h = jnp.maximum(
        jnp.dot(h, w1_ref[...], preferred_element_type=jnp.float32)
        + b1_ref[...], 0.0)
    o_ref[...] = jnp.tanh(
        jnp.dot(h, w2_ref[...], preferred_element_type=jnp.float32)
        + b2_ref[...])


def _mlp(x, part, W1, b1, W2, b2):
    return pl.pallas_call(
        _mlp_body,
        grid=(pl.cdiv(N, MB),),
        in_specs=[
            pl.BlockSpec((MB, D), lambda i: (i, 0)),
            pl.BlockSpec((NC, MB, D), lambda i: (0, i, 0)),
            pl.BlockSpec((D, D), lambda i: (0, 0)),
            pl.BlockSpec((1, D), lambda i: (0, 0)),
            pl.BlockSpec((D, D), lambda i: (0, 0)),
            pl.BlockSpec((1, D), lambda i: (0, 0)),
        ],
        out_specs=pl.BlockSpec((MB, D), lambda i: (i, 0)),
        out_shape=jax.ShapeDtypeStruct((N, D), jnp.float32),
    )(x, part, W1, b1, W2, b2)


def kernel(x, edge_index, W1, b1, W2, b2):
    pad = jnp.concatenate(
        [jnp.zeros((1, EP - E), jnp.int32),
         jnp.full((1, EP - E), N, jnp.int32)])
    ei = jnp.concatenate([edge_index, pad], axis=1)
    ei = ei.reshape(2, NW, SB, CB, K)
    zrows = jnp.zeros((RPT, D), jnp.float32)
    part = _sc_segment_sum(x, ei, zrows)
    return _mlp(x, part, W1, b1.reshape(1, D), W2, b2.reshape(1, D))


# speedup vs baseline: 2.9323x; 2.9323x over previous
import functools

import jax
import jax.numpy as jnp
from jax import lax
from jax.experimental import pallas as pl
from jax.experimental.pallas import tpu as pltpu
from jax.experimental.pallas import tpu_sc as plsc

N = 10000
E = 320000
D = 128

NC = 2
NS = 16
NW = NC * NS
K = 80
CB = 25
SB = 5
NP = 10240
RPT = NP // NS


def _sc_segment_sum(x, ei, zrows):
    mesh = plsc.VectorSubcoreMesh(core_axis_name="c", subcore_axis_name="s")

    @functools.partial(
        pl.kernel,
        out_type=jax.ShapeDtypeStruct((NC, NP, D), jnp.float32),
        mesh=mesh,
        scratch_types=[
            pltpu.VMEM((CB, K), jnp.int32),
            pltpu.VMEM((CB, K), jnp.int32),
            pltpu.VMEM((2, K, D), jnp.float32),
            pltpu.VMEM_SHARED((NP, D), jnp.float32),
            pltpu.SemaphoreType.DMA,
            pltpu.SemaphoreType.DMA,
        ],
    )
    def agg_kernel(x_hbm, ei_hbm, z_hbm, out_hbm, srcv, dstv, buf, agg_sh,
                   gsem, ssem):
        cid = lax.axis_index("c")
        sid = lax.axis_index("s")
        wid = cid * NS + sid
        base = sid * RPT
        pltpu.sync_copy(z_hbm, agg_sh.at[pl.ds(base, RPT)])
        plsc.subcore_barrier()

        def superblock(s, carry):
            pltpu.sync_copy(ei_hbm.at[0, wid, s], srcv)
            pltpu.sync_copy(ei_hbm.at[1, wid, s], dstv)
            pltpu.async_copy(x_hbm.at[srcv.at[0]], buf.at[0], gsem)

            def step(j, c):
                slot = lax.rem(j, 2)
                nslot = lax.rem(j + 1, 2)
                pltpu.make_async_copy(x_hbm.at[srcv.at[j]], buf.at[slot],
                                      gsem).wait()
                pltpu.async_copy(buf.at[slot], agg_sh.at[dstv.at[j]], ssem,
                                 add=True)

                @pl.when(j > 0)
                def _():
                    pltpu.make_async_copy(buf.at[nslot],
                                          agg_sh.at[dstv.at[j - 1]],
                                          ssem).wait()

                @pl.when(j < CB - 1)
                def _():
                    pltpu.async_copy(x_hbm.at[srcv.at[j + 1]],
                                     buf.at[nslot], gsem)

                return c

            lax.fori_loop(0, CB, step, 0)
            pltpu.make_async_copy(buf.at[(CB - 1) % 2],
                                  agg_sh.at[dstv.at[CB - 1]], ssem).wait()
            return carry

        lax.fori_loop(0, SB, superblock, 0)
        plsc.subcore_barrier()
        pltpu.sync_copy(agg_sh.at[pl.ds(base, RPT)],
                        out_hbm.at[cid, pl.ds(base, RPT)])

    return agg_kernel(x, ei, zrows)


MB = 512


def _mlp_body(x_ref, p_ref, w1_ref, b1_ref, w2_ref, b2_ref, o_ref):
    h = x_ref[...] + p_ref[0] + p_ref[1]
    h = jnp.maximum(
        jnp.dot(h, w1_ref[...], preferred_element_type=jnp.float32)
        + b1_ref[...], 0.0)
    o_ref[...] = jnp.tanh(
        jnp.dot(h, w2_ref[...], preferred_element_type=jnp.float32)
        + b2_ref[...])


def _mlp(x, part, W1, b1, W2, b2):
    return pl.pallas_call(
        _mlp_body,
        grid=(pl.cdiv(N, MB),),
        in_specs=[
            pl.BlockSpec((MB, D), lambda i: (i, 0)),
            pl.BlockSpec((NC, MB, D), lambda i: (0, i, 0)),
            pl.BlockSpec((D, D), lambda i: (0, 0)),
            pl.BlockSpec((1, D), lambda i: (0, 0)),
            pl.BlockSpec((D, D), lambda i: (0, 0)),
            pl.BlockSpec((1, D), lambda i: (0, 0)),
        ],
        out_specs=pl.BlockSpec((MB, D), lambda i: (i, 0)),
        out_shape=jax.ShapeDtypeStruct((N, D), jnp.float32),
    )(x, part, W1, b1, W2, b2)


def kernel(x, edge_index, W1, b1, W2, b2):
    ei = edge_index.reshape(2, NW, SB, CB, K)
    zrows = jnp.zeros((RPT, D), jnp.float32)
    part = _sc_segment_sum(x, ei, zrows)
    return _mlp(x, part, W1, b1.reshape(1, D), W2, b2.reshape(1, D))
